# Initial kernel scaffold; baseline (speedup 1.0000x reference)
import jax
import jax.numpy as jnp
from jax.experimental import pallas as pl


def kernel(x, gw1, gb1, gw2, gb2, gfw1, gfb1, gfw2, gfb2, ew1, eb1, ew2, eb2, efw1, efb1, efw2, efb2):
    raise NotImplementedError("write your pallas kernel here")


# speedup vs baseline: 2.0242x; 2.0242x over previous
import jax
import jax.numpy as jnp
from jax.experimental import pallas as pl
from jax.experimental.pallas import tpu as pltpu

E = 8
TOPK = 2
B = 1024
T = 128
NT_G = B // T
A = B * TOPK + E * T
NT_E = A // T


def _cnn_body(xg, w1, b1, w2, b2, fw1, fb1, fw2, fb2):
    x3 = xg.reshape(28, 28, T)
    xp = jnp.pad(x3, ((2, 2), (2, 2), (0, 0)))
    col = jnp.stack(
        [xp[dy:dy + 28, dx:dx + 28, :] for dy in range(5) for dx in range(5)],
        axis=0)
    h = jax.lax.dot_general(w1, col, (((1,), (0,)), ((), ())),
                            preferred_element_type=jnp.float32)
    h = jnp.maximum(h + b1[:, :, None, None], 0.0)
    h = h.reshape(32, 28, 14, 2, T)
    h = jnp.maximum(h[:, :, :, 0, :], h[:, :, :, 1, :])
    h = h.reshape(32, 14, 2, 14, T)
    h = jnp.maximum(h[:, :, 0, :, :], h[:, :, 1, :, :])

    hp = jnp.pad(h, ((0, 0), (2, 2), (2, 2), (0, 0)))
    acc = None
    for dy in range(5):
        chunk = jnp.stack([hp[:, dy:dy + 14, dx:dx + 14, :] for dx in range(5)],
                          axis=0)
        chunk = chunk.reshape(160, 14, 14, T)
        p = jax.lax.dot_general(w2[:, dy * 160:(dy + 1) * 160], chunk,
                                (((1,), (0,)), ((), ())),
                                preferred_element_type=jnp.float32)
        acc = p if acc is None else acc + p
    h2 = jnp.maximum(acc + b2[:, :, None, None], 0.0)
    h2 = h2.reshape(64, 14, 7, 2, T)
    h2 = jnp.maximum(h2[:, :, :, 0, :], h2[:, :, :, 1, :])
    h2 = h2.reshape(64, 7, 2, 7, T)
    h2 = jnp.maximum(h2[:, :, 0, :, :], h2[:, :, 1, :, :])

    flat = h2.reshape(3136, T)
    f1 = jnp.dot(fw1, flat, preferred_element_type=jnp.float32) + fb1
    f1 = jnp.maximum(f1, 0.0)
    f2 = jnp.dot(fw2, f1, preferred_element_type=jnp.float32) + fb2
    m = jnp.max(f2, axis=0, keepdims=True)
    ex = jnp.exp(f2 - m)
    return ex / jnp.sum(ex, axis=0, keepdims=True)


def _gate_kernel(xt_ref, w1_ref, b1_ref, w2_ref, b2_ref, fw1_ref, fb1_ref,
                 fw2_ref, fb2_ref, out_ref):
    probs = _cnn_body(xt_ref[...], w1_ref[...], b1_ref[...], w2_ref[...],
                      b2_ref[...], fw1_ref[...], fb1_ref[...], fw2_ref[...],
                      fb2_ref[...])
    m1 = jnp.max(probs, axis=0, keepdims=True)
    is1 = probs == m1
    pm = jnp.where(is1, -1e30, probs)
    m2 = jnp.max(pm, axis=0, keepdims=True)
    is2 = pm == m2
    denom = m1 + m2 + 1e-6
    iota_e = jax.lax.broadcasted_iota(jnp.int32, (8, T), 0).astype(jnp.float32)
    i1 = jnp.sum(jnp.where(is1, iota_e, 0.0), axis=0, keepdims=True)
    i2 = jnp.sum(jnp.where(is2, iota_e, 0.0), axis=0, keepdims=True)
    out_ref[...] = jnp.concatenate(
        [i1, i2, m1 / denom, m2 / denom, jnp.zeros((4, probs.shape[1]),
                                                   jnp.float32)], axis=0)


def _dispatch_kernel(nu_ref, tok_ref, xt_ref, out_ref):
    i = pl.program_id(0)

    @pl.when(i < nu_ref[0])
    def _():
        tok = tok_ref[0]
        iota_b = jax.lax.broadcasted_iota(jnp.int32, (B, T), 0)
        oh = jnp.where(iota_b == tok, 1.0, 0.0)
        out_ref[...] = jnp.dot(xt_ref[...], oh,
                               preferred_element_type=jnp.float32)

    @pl.when(i >= nu_ref[0])
    def _():
        out_ref[...] = jnp.zeros_like(out_ref)


def _expert_kernel(te_ref, nu_ref, xg_ref, w1_ref, b1_ref, w2_ref, b2_ref,
                   fw1_ref, fb1_ref, fw2_ref, fb2_ref, out_ref):
    i = pl.program_id(0)

    @pl.when(i < nu_ref[0])
    def _():
        out_ref[...] = _cnn_body(xg_ref[...], w1_ref[0], b1_ref[0], w2_ref[0],
                                 b2_ref[0], fw1_ref[0], fb1_ref[0], fw2_ref[0],
                                 fb2_ref[0])

    @pl.when(i >= nu_ref[0])
    def _():
        out_ref[...] = jnp.zeros_like(out_ref)


def _combine_kernel(eo_ref, tok_ref, w_ref, out_ref):
    iota_a = jax.lax.broadcasted_iota(jnp.int32, (A, B), 1)
    oh = jnp.where(iota_a == tok_ref[...], w_ref[...], 0.0)
    out_ref[...] = jnp.dot(eo_ref[...], oh, preferred_element_type=jnp.float32)


@jax.jit
def kernel(x, gw1, gb1, gw2, gb2, gfw1, gfb1, gfw2, gfb2, ew1, eb1, ew2, eb2,
           efw1, efb1, efw2, efb2):
    f32 = jnp.float32
    x_t = x.reshape(B, 784).T

    gw1r = gw1.reshape(32, 25)
    gw2r = gw2.transpose(0, 2, 3, 1).reshape(64, 800)
    gb1c = gb1.reshape(32, 1)
    gb2c = gb2.reshape(64, 1)
    gfb1c = gfb1.reshape(128, 1)
    gfb2c = gfb2.reshape(8, 1)

    ew1r = ew1.reshape(E, 32, 25)
    ew2r = ew2.transpose(0, 1, 3, 4, 2).reshape(E, 64, 800)
    eb1c = eb1.reshape(E, 32, 1)
    eb2c = eb2.reshape(E, 64, 1)
    efb1c = efb1.reshape(E, 128, 1)
    efw2p = jnp.concatenate([efw2, jnp.zeros((E, 6, 128), f32)], axis=1)
    efb2p = jnp.concatenate([efb2, jnp.full((E, 6), -1e30, f32)],
                            axis=1).reshape(E, 16, 1)

    full = lambda shp: pl.BlockSpec(shp, lambda i: tuple(0 for _ in shp))
    aux = pl.pallas_call(
        _gate_kernel,
        grid=(NT_G,),
        in_specs=[
            pl.BlockSpec((784, T), lambda i: (0, i)),
            full((32, 25)), full((32, 1)), full((64, 800)), full((64, 1)),
            full((128, 3136)), full((128, 1)), full((8, 128)), full((8, 1)),
        ],
        out_specs=pl.BlockSpec((8, T), lambda i: (0, i)),
        out_shape=jax.ShapeDtypeStruct((8, B), f32),
    )(x_t, gw1r, gb1c, gw2r, gb2c, gfw1, gfb1c, gfw2, gfb2c)

    i1 = aux[0].astype(jnp.int32)
    i2 = aux[1].astype(jnp.int32)
    wv1 = aux[2]
    wv2 = aux[3]

    experts = jnp.concatenate([i1, i2])
    tokens = jnp.concatenate([jnp.arange(B, dtype=jnp.int32)] * 2)
    wflat = jnp.concatenate([wv1, wv2])
    order = jnp.argsort(experts)
    es = experts[order]
    ts = tokens[order]
    ws = wflat[order]
    counts = jnp.zeros((E,), jnp.int32).at[es].add(1)
    start = jnp.concatenate([jnp.zeros((1,), jnp.int32),
                             jnp.cumsum(counts)[:-1]])
    padded = ((counts + T - 1) // T) * T
    pad_cum = jnp.cumsum(padded)
    pad_start = jnp.concatenate([jnp.zeros((1,), jnp.int32), pad_cum[:-1]])
    pos = pad_start[es] + jnp.arange(2 * B, dtype=jnp.int32) - start[es]
    tok_p = jnp.zeros((A,), jnp.int32).at[pos].set(ts)
    w_p = jnp.zeros((A,), f32).at[pos].set(ws)
    tile_expert = jnp.minimum(
        jnp.searchsorted(pad_cum, jnp.arange(NT_E, dtype=jnp.int32) * T,
                         side='right'), E - 1).astype(jnp.int32)
    nused = (pad_cum[-1] // T).astype(jnp.int32).reshape(1)

    xg_all = pl.pallas_call(
        _dispatch_kernel,
        grid_spec=pltpu.PrefetchScalarGridSpec(
            num_scalar_prefetch=1,
            grid=(NT_E,),
            in_specs=[
                pl.BlockSpec((1, 1, T), lambda i, nu: (i, 0, 0)),
                pl.BlockSpec((784, B), lambda i, nu: (0, 0)),
            ],
            out_specs=pl.BlockSpec((784, T), lambda i, nu: (0, i)),
        ),
        out_shape=jax.ShapeDtypeStruct((784, A), f32),
    )(nused, tok_p.reshape(NT_E, 1, T), x_t)

    eo = pl.pallas_call(
        _expert_kernel,
        grid_spec=pltpu.PrefetchScalarGridSpec(
            num_scalar_prefetch=2,
            grid=(NT_E,),
            in_specs=[
                pl.BlockSpec((784, T), lambda i, te, nu: (0, i)),
                pl.BlockSpec((1, 32, 25), lambda i, te, nu: (te[i], 0, 0)),
                pl.BlockSpec((1, 32, 1), lambda i, te, nu: (te[i], 0, 0)),
                pl.BlockSpec((1, 64, 800), lambda i, te, nu: (te[i], 0, 0)),
                pl.BlockSpec((1, 64, 1), lambda i, te, nu: (te[i], 0, 0)),
                pl.BlockSpec((1, 128, 3136), lambda i, te, nu: (te[i], 0, 0)),
                pl.BlockSpec((1, 128, 1), lambda i, te, nu: (te[i], 0, 0)),
                pl.BlockSpec((1, 16, 128), lambda i, te, nu: (te[i], 0, 0)),
                pl.BlockSpec((1, 16, 1), lambda i, te, nu: (te[i], 0, 0)),
            ],
            out_specs=pl.BlockSpec((16, T), lambda i, te, nu: (0, i)),
        ),
        out_shape=jax.ShapeDtypeStruct((16, A), f32),
    )(tile_expert, nused, xg_all, ew1r, eb1c, ew2r, eb2c, efw1, efb1c, efw2p,
      efb2p)

    out16 = pl.pallas_call(
        _combine_kernel,
        out_shape=jax.ShapeDtypeStruct((16, B), f32),
    )(eo, tok_p.reshape(A, 1), w_p.reshape(A, 1))

    return out16[:10].T
